# baseline (speedup 1.0000x reference)
import jax
import jax.numpy as jnp
from jax.experimental import pallas as pl
from jax.experimental.pallas import tpu as pltpu

_B = 4
_L = 1024
_H = 4
_D = 32
_BS = 32
_S = 4
_T = _B * _L
_NB = _L // _BS
_SCALE = _D ** (-0.5)


def _silu(x):
    return x * jax.nn.sigmoid(x)


_TW = 512
_NW = _L // _TW


def _fwd(q_ref, k_ref, v_ref, gc_ref, gs_ref, o_ref):
    i = pl.program_id(2)
    qt = q_ref[0, 0]
    kk = k_ref[0, 0]
    vv = v_ref[0, 0]
    gc = gc_ref[0, 0, 0]
    gs = gs_ref[0, 0, 0]

    blk_row = jax.lax.broadcasted_iota(jnp.int32, (_NB, _TW), 0)

    k_cmp = kk.reshape(_NB, _BS, _D).sum(axis=1) * (1.0 / _BS)
    v_cmp = vv.reshape(_NB, _BS, _D).sum(axis=1) * (1.0 / _BS)

    s_cmp = jax.lax.dot_general(
        k_cmp, qt, (((1,), (1,)), ((), ())),
        preferred_element_type=jnp.float32) * _SCALE
    tok_of_col = (i * _TW
                  + jax.lax.broadcasted_iota(jnp.int32, (_NB, _TW), 1)) // _BS
    blk_causal = tok_of_col >= blk_row
    p_cmp = jnp.where(blk_causal, _silu(s_cmp), 0.0)
    o_cmp = jax.lax.dot_general(
        p_cmp, v_cmp, (((0,), (0,)), ((), ())),
        preferred_element_type=jnp.float32) * gc

    neginf = jnp.float32(-jnp.inf)
    work = jnp.where(blk_causal, s_cmp, neginf)
    sel = jnp.zeros((_NB, _TW), dtype=jnp.bool_)
    for _ in range(_S):
        m = jnp.max(work, axis=0, keepdims=True)
        ismax = jnp.logical_and(work == m, jnp.logical_not(sel))
        cand = jnp.where(ismax, blk_row, _NB)
        mi = jnp.min(cand, axis=0, keepdims=True)
        pick = blk_row == mi
        sel = jnp.logical_or(sel, pick)
        work = jnp.where(pick, neginf, work)

    qts = qt * _SCALE

    def _slc_branch(nk):
        kj = kk[:nk]
        s_full = jax.lax.dot_general(
            kj, qts, (((1,), (1,)), ((), ())),
            preferred_element_type=jnp.float32)
        sel_rep = jnp.repeat(sel[:nk // _BS], _BS, axis=0)
        rowj = jax.lax.broadcasted_iota(jnp.int32, (nk, _TW), 0)
        coll = i * _TW + jax.lax.broadcasted_iota(jnp.int32, (nk, _TW), 1)
        keep = jnp.logical_and(sel_rep, rowj <= coll)
        p = jnp.where(keep, _silu(s_full), 0.0)
        return jax.lax.dot_general(
            p, vv[:nk], (((0,), (0,)), ((), ())),
            preferred_element_type=jnp.float32) * gs

    for iw in range(_NW):
        @pl.when(i == iw)
        def _():
            o_ref[0, 0] = o_cmp + _slc_branch((iw + 1) * _TW)


def _run(qh, kh, vh, gc, gs, interpret=False):
    return pl.pallas_call(
        _fwd,
        grid=(_B, _H, _NW),
        in_specs=[
            pl.BlockSpec((1, 1, _TW, _D), lambda b, h, i: (b, h, i, 0)),
            pl.BlockSpec((1, 1, _L, _D), lambda b, h, i: (b, h, 0, 0)),
            pl.BlockSpec((1, 1, _L, _D), lambda b, h, i: (b, h, 0, 0)),
            pl.BlockSpec((1, 1, 1, _TW, 1), lambda b, h, i: (b, h, 0, i, 0)),
            pl.BlockSpec((1, 1, 1, _TW, 1), lambda b, h, i: (b, h, 0, i, 0)),
        ],
        out_specs=pl.BlockSpec((1, 1, _TW, _D), lambda b, h, i: (b, h, i, 0)),
        out_shape=jax.ShapeDtypeStruct((_B, _H, _L, _D), jnp.float32),
        compiler_params=pltpu.CompilerParams(
            dimension_semantics=("parallel", "parallel", "arbitrary")),
        interpret=interpret,
    )(qh, kh, vh, gc, gs)


def kernel(q, k, v, g_cmp, g_slc, x_offsets):
    del x_offsets
    qh = q.reshape(_B, _L, _H, _D).transpose(0, 2, 1, 3)
    kh = k.reshape(_B, _L, _H, _D).transpose(0, 2, 1, 3)
    vh = v.reshape(_B, _L, _H, _D).transpose(0, 2, 1, 3)
    gc = g_cmp.reshape(_B, _L, _H).transpose(0, 2, 1).reshape(_B, _H, 1, _L, 1)
    gs = g_slc.reshape(_B, _L, _H).transpose(0, 2, 1).reshape(_B, _H, 1, _L, 1)
    out = _run(qh, kh, vh, gc, gs)
    return out.transpose(0, 2, 1, 3).reshape(_T, _H, _D)


# speedup vs baseline: 1.1419x; 1.1419x over previous
import jax
import jax.numpy as jnp
from jax.experimental import pallas as pl
from jax.experimental.pallas import tpu as pltpu

_B = 4
_L = 1024
_H = 4
_D = 32
_BS = 32
_S = 4
_T = _B * _L
_NB = _L // _BS
_SCALE = _D ** (-0.5)


def _silu(x):
    return x * jax.nn.sigmoid(x)


_TW = _L
_NW = _L // _TW


def _fwd(q_ref, k_ref, v_ref, gc_ref, gs_ref, o_ref):
    i = pl.program_id(2)
    qt = q_ref[0, 0]
    kk = k_ref[0, 0]
    vv = v_ref[0, 0]
    gc = gc_ref[0, 0, 0]
    gs = gs_ref[0, 0, 0]

    blk_row = jax.lax.broadcasted_iota(jnp.int32, (_NB, _TW), 0)

    k_cmp = kk.reshape(_NB, _BS, _D).sum(axis=1) * (1.0 / _BS)
    v_cmp = vv.reshape(_NB, _BS, _D).sum(axis=1) * (1.0 / _BS)

    s_cmp = jax.lax.dot_general(
        k_cmp, qt, (((1,), (1,)), ((), ())),
        preferred_element_type=jnp.float32) * _SCALE
    tok_of_col = (i * _TW
                  + jax.lax.broadcasted_iota(jnp.int32, (_NB, _TW), 1)) // _BS
    blk_causal = tok_of_col >= blk_row
    p_cmp = jnp.where(blk_causal, _silu(s_cmp), 0.0)
    o_cmp = jax.lax.dot_general(
        p_cmp, v_cmp, (((0,), (0,)), ((), ())),
        preferred_element_type=jnp.float32) * gc

    neginf = jnp.float32(-jnp.inf)
    work = jnp.where(blk_causal, s_cmp, neginf)
    sel = jnp.zeros((_NB, _TW), dtype=jnp.bool_)
    for _ in range(_S):
        m = jnp.max(work, axis=0, keepdims=True)
        ismax = jnp.logical_and(work == m, jnp.logical_not(sel))
        cand = jnp.where(ismax, blk_row, _NB)
        mi = jnp.min(cand, axis=0, keepdims=True)
        pick = blk_row == mi
        sel = jnp.logical_or(sel, pick)
        work = jnp.where(pick, neginf, work)

    qts = qt * _SCALE

    def _slc_branch(nk):
        kj = kk[:nk]
        s_full = jax.lax.dot_general(
            kj, qts, (((1,), (1,)), ((), ())),
            preferred_element_type=jnp.float32)
        sel_rep = jnp.repeat(sel[:nk // _BS], _BS, axis=0)
        rowj = jax.lax.broadcasted_iota(jnp.int32, (nk, _TW), 0)
        coll = i * _TW + jax.lax.broadcasted_iota(jnp.int32, (nk, _TW), 1)
        keep = jnp.logical_and(sel_rep, rowj <= coll)
        p = jnp.where(keep, _silu(s_full), 0.0)
        return jax.lax.dot_general(
            p, vv[:nk], (((0,), (0,)), ((), ())),
            preferred_element_type=jnp.float32) * gs

    for iw in range(_NW):
        @pl.when(i == iw)
        def _():
            o_ref[0, 0] = o_cmp + _slc_branch((iw + 1) * _TW)


def _run(qh, kh, vh, gc, gs, interpret=False):
    return pl.pallas_call(
        _fwd,
        grid=(_B, _H, _NW),
        in_specs=[
            pl.BlockSpec((1, 1, _TW, _D), lambda b, h, i: (b, h, i, 0)),
            pl.BlockSpec((1, 1, _L, _D), lambda b, h, i: (b, h, 0, 0)),
            pl.BlockSpec((1, 1, _L, _D), lambda b, h, i: (b, h, 0, 0)),
            pl.BlockSpec((1, 1, 1, _TW, 1), lambda b, h, i: (b, h, 0, i, 0)),
            pl.BlockSpec((1, 1, 1, _TW, 1), lambda b, h, i: (b, h, 0, i, 0)),
        ],
        out_specs=pl.BlockSpec((1, 1, _TW, _D), lambda b, h, i: (b, h, i, 0)),
        out_shape=jax.ShapeDtypeStruct((_B, _H, _L, _D), jnp.float32),
        compiler_params=pltpu.CompilerParams(
            dimension_semantics=("parallel", "parallel", "arbitrary")),
        interpret=interpret,
    )(qh, kh, vh, gc, gs)


def kernel(q, k, v, g_cmp, g_slc, x_offsets):
    del x_offsets
    qh = q.reshape(_B, _L, _H, _D).transpose(0, 2, 1, 3)
    kh = k.reshape(_B, _L, _H, _D).transpose(0, 2, 1, 3)
    vh = v.reshape(_B, _L, _H, _D).transpose(0, 2, 1, 3)
    gc = g_cmp.reshape(_B, _L, _H).transpose(0, 2, 1).reshape(_B, _H, 1, _L, 1)
    gs = g_slc.reshape(_B, _L, _H).transpose(0, 2, 1).reshape(_B, _H, 1, _L, 1)
    out = _run(qh, kh, vh, gc, gs)
    return out.transpose(0, 2, 1, 3).reshape(_T, _H, _D)
